# baseline (speedup 1.0000x reference)
import jax
import jax.numpy as jnp
from jax.experimental import pallas as pl
from jax.experimental.pallas import tpu as pltpu

_ROWS = 4 * 2048
_COLS = 1024
_K = 8


def kernel(input1, output1):
    rows_per = _ROWS // _K

    def body(in_ref, out_ref, *sems):
        copies = []
        for i in range(_K):
            copies.append(
                pltpu.make_async_copy(
                    in_ref.at[pl.ds(i * rows_per, rows_per)],
                    out_ref.at[pl.ds(i * rows_per, rows_per)],
                    sems[i],
                )
            )
        for c in copies:
            c.start()
        for c in copies:
            c.wait()

    out = pl.pallas_call(
        body,
        in_specs=[pl.BlockSpec(memory_space=pl.ANY)],
        out_specs=pl.BlockSpec(memory_space=pl.ANY),
        out_shape=jax.ShapeDtypeStruct((_ROWS, _COLS), jnp.float32),
        scratch_shapes=[pltpu.SemaphoreType.DMA] * _K,
    )(input1.reshape(_ROWS, _COLS))
    return out.reshape(input1.shape).astype(output1.dtype)


# speedup vs baseline: 41.6809x; 41.6809x over previous
import jax
import jax.numpy as jnp
from jax.experimental import pallas as pl
from jax.experimental.pallas import tpu as pltpu

_ROWS = 4 * 2048
_COLS = 1024
_BLK = 512


def kernel(input1, output1):
    def body(in_ref, out_ref):
        out_ref[...] = in_ref[...]

    out = pl.pallas_call(
        body,
        grid=(_ROWS // _BLK,),
        in_specs=[pl.BlockSpec((_BLK, _COLS), lambda i: (i, 0))],
        out_specs=pl.BlockSpec((_BLK, _COLS), lambda i: (i, 0)),
        out_shape=jax.ShapeDtypeStruct((_ROWS, _COLS), jnp.float32),
        compiler_params=pltpu.CompilerParams(
            dimension_semantics=("parallel",),
        ),
    )(input1.reshape(_ROWS, _COLS))
    return out.reshape(input1.shape).astype(output1.dtype)
